# Initial kernel scaffold; baseline (speedup 1.0000x reference)
import jax
import jax.numpy as jnp
from jax.experimental import pallas as pl


def kernel(x, W, b):
    raise NotImplementedError("write your pallas kernel here")


# speedup vs baseline: 1.9693x; 1.9693x over previous
import functools

import jax
import jax.numpy as jnp
from jax.experimental import pallas as pl

TOKENS = 16384
INPUT_DIM = 2048
NUM_EXPERTS = 16
TILE = 512


def _gating_body(x_ref, w_ref, b_ref, wts_ref, idx_ref, mask_ref, probs_ref):
    logits = jnp.dot(x_ref[...], w_ref[...],
                     preferred_element_type=jnp.float32) + b_ref[...]
    m = jnp.max(logits, axis=1, keepdims=True)
    e = jnp.exp(logits - m)
    p = e / jnp.sum(e, axis=1, keepdims=True)
    probs_ref[...] = p

    lane = jax.lax.broadcasted_iota(jnp.int32, p.shape, 1)
    m1 = jnp.max(p, axis=1, keepdims=True)
    i1 = jnp.min(jnp.where(p == m1, lane, NUM_EXPERTS), axis=1, keepdims=True)
    p2 = jnp.where(lane == i1, -1.0, p)
    m2 = jnp.max(p2, axis=1, keepdims=True)
    i2 = jnp.min(jnp.where(p2 == m2, lane, NUM_EXPERTS), axis=1, keepdims=True)

    ssum = m1 + m2
    wts_ref[...] = jnp.concatenate([m1 / ssum, m2 / ssum], axis=1)
    idx_ref[...] = jnp.concatenate([i1, i2], axis=1)
    mask_ref[...] = ((lane == i1) | (lane == i2)).astype(jnp.float32)


@functools.partial(jax.jit, static_argnames=("interpret",))
def kernel(x, W, b, interpret=False):
    n_tiles = TOKENS // TILE
    grid = (n_tiles,)
    out_shapes = (
        jax.ShapeDtypeStruct((TOKENS, 2), jnp.float32),
        jax.ShapeDtypeStruct((TOKENS, 2), jnp.int32),
        jax.ShapeDtypeStruct((TOKENS, NUM_EXPERTS), jnp.float32),
        jax.ShapeDtypeStruct((TOKENS, NUM_EXPERTS), jnp.float32),
    )
    in_specs = [
        pl.BlockSpec((TILE, INPUT_DIM), lambda i: (i, 0)),
        pl.BlockSpec((INPUT_DIM, NUM_EXPERTS), lambda i: (0, 0)),
        pl.BlockSpec((1, NUM_EXPERTS), lambda i: (0, 0)),
    ]
    out_specs = (
        pl.BlockSpec((TILE, 2), lambda i: (i, 0)),
        pl.BlockSpec((TILE, 2), lambda i: (i, 0)),
        pl.BlockSpec((TILE, NUM_EXPERTS), lambda i: (i, 0)),
        pl.BlockSpec((TILE, NUM_EXPERTS), lambda i: (i, 0)),
    )
    return pl.pallas_call(
        _gating_body,
        grid=grid,
        in_specs=in_specs,
        out_specs=out_specs,
        out_shape=out_shapes,
        interpret=interpret,
    )(x, W, b.reshape(1, NUM_EXPERTS))
